# baseline (speedup 1.0000x reference)
import functools

import jax
import jax.numpy as jnp
from jax import lax
from jax.experimental import pallas as pl
from jax.experimental.pallas import tpu as pltpu
from jax.experimental.pallas import tpu_sc as plsc

N = 10000
E = 320000
D = 128
OUT = 64

NC = 2
NS = 16
NW = NC * NS
CH = 128
NCHUNK = 80
HCHUNK = 40
E_W = NCHUNK * CH
E_PAD = E_W * NW
NP = 10240
ROWS_T = NP // NS
PAD_DST = N

_f32 = jnp.float32


def _sc_mesh():
    return plsc.VectorSubcoreMesh(
        core_axis_name="c", subcore_axis_name="s", num_cores=NC, num_subcores=NS
    )


def _fill_buf(buf, val):
    v16 = jnp.full((16,), val, _f32)

    def _row(i, _):
        for j in range(D // 16):
            buf[i, pl.ds(j * 16, 16)] = v16
        return 0

    lax.fori_loop(0, CH, _row, 0)


def _zero_my_rows(buf0, acc_sh, s, zsem):
    for k in range(ROWS_T // CH):
        pltpu.async_copy(buf0, acc_sh.at[pl.ds(s * ROWS_T + k * CH, CH)],
                         zsem)
    for k in range(ROWS_T // CH):
        pltpu.make_async_copy(
            buf0, acc_sh.at[pl.ds(s * ROWS_T + k * CH, CH)], zsem).wait()


def _seg_body(h, srcw, dstw, acc_out, src_v, dst_v, buf0, buf1, acc_sh,
              sem0, sem1, deg_out=None):
    c = lax.axis_index("c")
    s = lax.axis_index("s")
    wid = c * NS + s

    _fill_buf(buf0, 0.0)
    _zero_my_rows(buf0, acc_sh, s, sem1)
    plsc.subcore_barrier()

    if deg_out is not None:
        _fill_buf(buf0, 1.0)
        for half in range(NCHUNK // HCHUNK):
            pltpu.sync_copy(dstw.at[wid, pl.ds(half * HCHUNK, HCHUNK)],
                            dst_v)

            def _fire(j, _):
                pltpu.async_copy(buf0, acc_sh.at[dst_v.at[j]], sem1,
                                 add=True)
                return 0

            lax.fori_loop(0, HCHUNK, _fire, 0)

            def _drain(j, _):
                pltpu.make_async_copy(buf0, acc_sh.at[dst_v.at[j]],
                                      sem1).wait()
                return 0

            lax.fori_loop(0, HCHUNK, _drain, 0)
        plsc.subcore_barrier()
        pltpu.sync_copy(acc_sh.at[pl.ds(s * ROWS_T, ROWS_T)],
                        deg_out.at[c, pl.ds(s * ROWS_T, ROWS_T)])
        _fill_buf(buf0, 0.0)
        _zero_my_rows(buf0, acc_sh, s, sem1)
        plsc.subcore_barrier()

    for half in range(NCHUNK // HCHUNK):
        pltpu.sync_copy(srcw.at[wid, pl.ds(half * HCHUNK, HCHUNK)], src_v)
        pltpu.sync_copy(dstw.at[wid, pl.ds(half * HCHUNK, HCHUNK)], dst_v)

        pltpu.async_copy(h.at[src_v.at[0]], buf0, sem0)

        def _pair(i, _):
            j0 = 2 * i
            pltpu.async_copy(h.at[src_v.at[j0 + 1]], buf1, sem1)
            pltpu.make_async_copy(h.at[src_v.at[j0]], buf0, sem0).wait()
            pltpu.sync_copy(buf0, acc_sh.at[dst_v.at[j0]], add=True)

            @pl.when(j0 + 2 < HCHUNK)
            def _():
                pltpu.async_copy(h.at[src_v.at[j0 + 2]], buf0, sem0)

            pltpu.make_async_copy(h.at[src_v.at[j0 + 1]], buf1, sem1).wait()
            pltpu.sync_copy(buf1, acc_sh.at[dst_v.at[j0 + 1]], add=True)
            return 0

        lax.fori_loop(0, HCHUNK // 2, _pair, 0)
    plsc.subcore_barrier()

    pltpu.sync_copy(acc_sh.at[pl.ds(s * ROWS_T, ROWS_T)],
                    acc_out.at[c, pl.ds(s * ROWS_T, ROWS_T)])


def _seg_kernel(h, srcw, dstw):
    f = pl.kernel(
        _seg_body,
        out_type=jax.ShapeDtypeStruct((NC, NP, D), _f32),
        mesh=_sc_mesh(),
        scratch_types=[
            pltpu.VMEM((HCHUNK, CH), jnp.int32),
            pltpu.VMEM((HCHUNK, CH), jnp.int32),
            pltpu.VMEM((CH, D), _f32),
            pltpu.VMEM((CH, D), _f32),
            pltpu.VMEM_SHARED((NP, D), _f32),
            pltpu.SemaphoreType.DMA,
            pltpu.SemaphoreType.DMA,
        ],
    )
    return f(h, srcw, dstw)


def _seg0_deg_kernel(h, srcw, dstw):
    def body(h, srcw, dstw, acc_out, deg_out, src_v, dst_v, buf0, buf1,
             acc_sh, sem0, sem1):
        _seg_body(h, srcw, dstw, acc_out, src_v, dst_v, buf0, buf1, acc_sh,
                  sem0, sem1, deg_out=deg_out)

    f = pl.kernel(
        body,
        out_type=[jax.ShapeDtypeStruct((NC, NP, D), _f32),
                  jax.ShapeDtypeStruct((NC, NP, D), _f32)],
        mesh=_sc_mesh(),
        scratch_types=[
            pltpu.VMEM((HCHUNK, CH), jnp.int32),
            pltpu.VMEM((HCHUNK, CH), jnp.int32),
            pltpu.VMEM((CH, D), _f32),
            pltpu.VMEM((CH, D), _f32),
            pltpu.VMEM_SHARED((NP, D), _f32),
            pltpu.SemaphoreType.DMA,
            pltpu.SemaphoreType.DMA,
        ],
    )
    return f(h, srcw, dstw)


R = 5000
GRID = N // R

def _row_spec(width):
    return pl.BlockSpec((R, width), lambda i: (i, 0))


def _full_spec(shape):
    nd = len(shape)
    return pl.BlockSpec(shape, lambda i: (0,) * nd)


def _dot(a, b):
    return jnp.dot(a, b, preferred_element_type=_f32)


def _agg(a0, a1, d0, d1):
    deg = jnp.maximum(d0[:, :1] + d1[:, :1], 1.0)
    return (a0 + a1) / deg


def _conv_out(x, a_r, dg_r, ws, wn, b):
    agg = _agg(a_r[0], a_r[1], dg_r[0], dg_r[1])
    return jnp.maximum(_dot(x, ws) + _dot(agg, wn) + b, 0.0)


def _acc_spec():
    return pl.BlockSpec((NC, R, D), lambda i: (0, i, 0))


def _deg_spec():
    return pl.BlockSpec((NC, R, D), lambda i: (0, i, 0))


def _tc_call(body, n_out, widths_in, widths_out):
    return pl.pallas_call(
        body,
        grid=(GRID,),
        in_specs=[w if isinstance(w, pl.BlockSpec)
                  else (_row_spec(w) if isinstance(w, int) else _full_spec(w))
                  for w in widths_in],
        out_specs=[_row_spec(w) for w in widths_out] if n_out > 1
        else _row_spec(widths_out[0]),
        out_shape=[jax.ShapeDtypeStruct((N, w), _f32) for w in widths_out]
        if n_out > 1 else jax.ShapeDtypeStruct((N, widths_out[0]), _f32),
    )


def _k1(x, ip, t00):
    def body(x_r, w1, b1, g, be, w2, b2, w0, b0, o_r):
        h = jnp.maximum(_dot(x_r[...], w1[...]) + b1[...], 0.0)
        mu = jnp.mean(h, axis=-1, keepdims=True)
        var = jnp.mean(jnp.square(h - mu), axis=-1, keepdims=True)
        hn = (h - mu) * lax.rsqrt(var + 1e-5) * g[...] + be[...]
        t0 = _dot(hn, w2[...]) + b2[...]
        o_r[...] = jnp.maximum(_dot(t0, w0[...]) + b0[...], 0.0)

    mid = ip["l1"]["W"].shape[1]
    f = _tc_call(body, 1,
                 [D, (D, mid), (1, mid), (1, mid), (1, mid), (mid, D), (1, D),
                  (D, D), (1, D)], [D])
    return f(x, ip["l1"]["W"], ip["l1"]["b"].reshape(1, -1),
             ip["g"].reshape(1, -1), ip["be"].reshape(1, -1),
             ip["l2"]["W"], ip["l2"]["b"].reshape(1, -1),
             t00["W"], t00["b"].reshape(1, -1))


def _k2(x, acc, deg, cv):
    def body(x_r, a_r, dg_r, ws, wn, b, o_r):
        o_r[...] = _conv_out(x_r[...], a_r, dg_r, ws[...], wn[...], b[...])

    f = _tc_call(body, 1, [D, _acc_spec(), _deg_spec(), (D, D), (D, D),
                           (1, D)], [D])
    return f(x, acc, deg, cv["Ws"], cv["Wn"], cv["b"].reshape(1, -1))


def _k3(bot, acc, deg, cv, br0, t01):
    def body(bot_r, a_r, dg_r, ws, wn, b, br_r, wt, bt, o_r):
        cb = _conv_out(bot_r[...], a_r, dg_r, ws[...], wn[...], b[...])
        tb = jnp.maximum(_dot(br_r[...], wt[...]) + bt[...], 0.0)
        o_r[...] = (cb + tb) * 0.5

    f = _tc_call(body, 1, [D, _acc_spec(), _deg_spec(), (D, D), (D, D),
                           (1, D), D, (D, D), (1, D)], [D])
    return f(bot, acc, deg, cv["Ws"], cv["Wn"], cv["b"].reshape(1, -1),
             br0, t01["W"], t01["b"].reshape(1, -1))


def _k4(fin, acc, deg, cv, t02, t12):
    def body(f_r, a_r, dg_r, ws, wn, b, w0, b0, w1, b1,
             bot_r, o0_r, o1_r):
        fv = f_r[...]
        bot_r[...] = _conv_out(fv, a_r, dg_r, ws[...], wn[...], b[...])
        o0_r[...] = jnp.maximum(_dot(fv, w0[...]) + b0[...], 0.0)
        o1_r[...] = jnp.maximum(_dot(fv, w1[...]) + b1[...], 0.0)

    f = _tc_call(body, 3, [D, _acc_spec(), _deg_spec(), (D, D), (D, D),
                           (1, D), (D, D), (1, D), (D, D), (1, D)], [D, D, D])
    return f(fin, acc, deg, cv["Ws"], cv["Wn"], cv["b"].reshape(1, -1),
             t02["W"], t02["b"].reshape(1, -1),
             t12["W"], t12["b"].reshape(1, -1))


def _k5(bot, br0, br1, acc, deg, cv, t03, t13):
    def body(bot_r, br0_r, br1_r, a_r, dg_r, ws, wn, b,
             w0, b0, w1, b1, o_r):
        cb = _conv_out(bot_r[...], a_r, dg_r, ws[...], wn[...], b[...])
        tb0 = jnp.maximum(_dot(br0_r[...], w0[...]) + b0[...], 0.0)
        tb1 = jnp.maximum(_dot(br1_r[...], w1[...]) + b1[...], 0.0)
        o_r[...] = (cb + tb0 + tb1) * (1.0 / 3.0)

    f = _tc_call(body, 1, [D, D, D, _acc_spec(), _deg_spec(), (D, D),
                           (D, D), (1, D), (D, D), (1, D), (D, D), (1, D)],
                 [D])
    return f(bot, br0, br1, acc, deg,
             cv["Ws"], cv["Wn"], cv["b"].reshape(1, -1),
             t03["W"], t03["b"].reshape(1, -1),
             t13["W"], t13["b"].reshape(1, -1))


def _k6(f2, acc, deg, cv, t04, t14, t24, dp):
    def body(f_r, a_r, dg_r, ws, wn, b, w0, b0, w1, b1,
             w2, b2, wd, bd, o_r):
        fv = f_r[...]
        cb = _conv_out(fv, a_r, dg_r, ws[...], wn[...], b[...])
        tb0 = jnp.maximum(_dot(fv, w0[...]) + b0[...], 0.0)
        tb1 = jnp.maximum(_dot(fv, w1[...]) + b1[...], 0.0)
        tb2 = jnp.maximum(_dot(fv, w2[...]) + b2[...], 0.0)
        merged = (cb + tb0 + tb1 + tb2) * 0.25
        o_r[...] = _dot(merged, wd[...]) + bd[...]

    f = _tc_call(body, 1, [D, _acc_spec(), _deg_spec(), (D, D), (D, D),
                           (1, D), (D, D), (1, D), (D, D), (1, D),
                           (D, D), (1, D), (D, OUT), (1, OUT)], [OUT])
    return f(f2, acc, deg, cv["Ws"], cv["Wn"], cv["b"].reshape(1, -1),
             t04["W"], t04["b"].reshape(1, -1),
             t14["W"], t14["b"].reshape(1, -1),
             t24["W"], t24["b"].reshape(1, -1),
             dp["W"], dp["b"].reshape(1, -1))


def _prep_edges(edge_index):
    src = edge_index[0]
    dst = edge_index[1]
    pad = E_PAD - E
    ar = jnp.arange(pad, dtype=jnp.int32)
    srcp = jnp.concatenate([src, ar % N])
    dstp = jnp.concatenate([dst, PAD_DST + ar % (NP - N)])
    return (srcp.reshape(NW, NCHUNK, CH), dstp.reshape(NW, NCHUNK, CH))


def kernel(x, edge_index, params):
    conv = params["conv"]
    t = params["t"]
    srcw, dstw = _prep_edges(edge_index)

    br0 = _k1(x, params["ip"], t["0_0"])
    acc, deg = _seg0_deg_kernel(x, srcw, dstw)
    bot = _k2(x, acc, deg, conv[0])

    acc = _seg_kernel(bot, srcw, dstw)
    f = _k3(bot, acc, deg, conv[1], br0, t["0_1"])

    acc = _seg_kernel(f, srcw, dstw)
    bot, br0, br1 = _k4(f, acc, deg, conv[2], t["0_2"], t["1_2"])

    acc = _seg_kernel(bot, srcw, dstw)
    f2 = _k5(bot, br0, br1, acc, deg, conv[3], t["0_3"], t["1_3"])

    acc = _seg_kernel(f2, srcw, dstw)
    return _k6(f2, acc, deg, conv[4], t["0_4"], t["1_4"], t["2_4"],
               params["dp"])


# speedup vs baseline: 1.0075x; 1.0075x over previous
import jax
import jax.numpy as jnp
from jax import lax
from jax.experimental import pallas as pl
from jax.experimental.pallas import tpu as pltpu
from jax.experimental.pallas import tpu_sc as plsc

N = 10000
E = 320000
D = 128
OUT = 64

NC = 2
NS = 16
NW = NC * NS
CH = 128
NCHUNK = 80
HCHUNK = 40
E_W = NCHUNK * CH
E_PAD = E_W * NW
NP = 10240
ROWS_T = NP // NS
PAD_DST = N

_f32 = jnp.float32


def _sc_mesh():
    return plsc.VectorSubcoreMesh(
        core_axis_name="c", subcore_axis_name="s", num_cores=NC, num_subcores=NS
    )


def _fill_buf(buf, val):
    v16 = jnp.full((16,), val, _f32)

    def _row(i, _):
        for j in range(D // 16):
            buf[i, pl.ds(j * 16, 16)] = v16
        return 0

    lax.fori_loop(0, CH, _row, 0)


def _zero_my_rows(buf0, acc_sh, s, zsem):
    for k in range(ROWS_T // CH):
        pltpu.async_copy(buf0, acc_sh.at[pl.ds(s * ROWS_T + k * CH, CH)],
                         zsem)
    for k in range(ROWS_T // CH):
        pltpu.make_async_copy(
            buf0, acc_sh.at[pl.ds(s * ROWS_T + k * CH, CH)], zsem).wait()


def _seg_body(h, srcw, dstw, acc_out, src_v, dst_v, buf0, buf1, acc_sh,
              sem0, sem1, deg_out=None):
    c = lax.axis_index("c")
    s = lax.axis_index("s")
    wid = c * NS + s

    _fill_buf(buf0, 0.0)
    _zero_my_rows(buf0, acc_sh, s, sem1)
    plsc.subcore_barrier()

    if deg_out is not None:
        _fill_buf(buf0, 1.0)
        for half in range(NCHUNK // HCHUNK):
            pltpu.sync_copy(dstw.at[wid, pl.ds(half * HCHUNK, HCHUNK)],
                            dst_v)

            def _fire(j, _):
                pltpu.async_copy(buf0, acc_sh.at[dst_v.at[j]], sem1,
                                 add=True)
                return 0

            lax.fori_loop(0, HCHUNK, _fire, 0)

            def _drain(j, _):
                pltpu.make_async_copy(buf0, acc_sh.at[dst_v.at[j]],
                                      sem1).wait()
                return 0

            lax.fori_loop(0, HCHUNK, _drain, 0)
        plsc.subcore_barrier()
        pltpu.sync_copy(acc_sh.at[pl.ds(s * ROWS_T, ROWS_T)],
                        deg_out.at[c, pl.ds(s * ROWS_T, ROWS_T)])
        _fill_buf(buf0, 0.0)
        _zero_my_rows(buf0, acc_sh, s, sem1)
        plsc.subcore_barrier()

    for half in range(NCHUNK // HCHUNK):
        pltpu.sync_copy(srcw.at[wid, pl.ds(half * HCHUNK, HCHUNK)], src_v)
        pltpu.sync_copy(dstw.at[wid, pl.ds(half * HCHUNK, HCHUNK)], dst_v)

        pltpu.async_copy(h.at[src_v.at[0]], buf0, sem0)

        def _pair(i, _):
            j0 = 2 * i
            pltpu.async_copy(h.at[src_v.at[j0 + 1]], buf1, sem1)
            pltpu.make_async_copy(h.at[src_v.at[j0]], buf0, sem0).wait()
            pltpu.sync_copy(buf0, acc_sh.at[dst_v.at[j0]], add=True)

            @pl.when(j0 + 2 < HCHUNK)
            def _():
                pltpu.async_copy(h.at[src_v.at[j0 + 2]], buf0, sem0)

            pltpu.make_async_copy(h.at[src_v.at[j0 + 1]], buf1, sem1).wait()
            pltpu.sync_copy(buf1, acc_sh.at[dst_v.at[j0 + 1]], add=True)
            return 0

        lax.fori_loop(0, HCHUNK // 2, _pair, 0)
    plsc.subcore_barrier()

    pltpu.sync_copy(acc_sh.at[pl.ds(s * ROWS_T, ROWS_T)],
                    acc_out.at[c, pl.ds(s * ROWS_T, ROWS_T)])


def _seg_kernel(h, srcw, dstw):
    f = pl.kernel(
        _seg_body,
        out_type=jax.ShapeDtypeStruct((NC, NP, D), _f32),
        mesh=_sc_mesh(),
        scratch_types=[
            pltpu.VMEM((HCHUNK, CH), jnp.int32),
            pltpu.VMEM((HCHUNK, CH), jnp.int32),
            pltpu.VMEM((CH, D), _f32),
            pltpu.VMEM((CH, D), _f32),
            pltpu.VMEM_SHARED((NP, D), _f32),
            pltpu.SemaphoreType.DMA,
            pltpu.SemaphoreType.DMA,
        ],
    )
    return f(h, srcw, dstw)


def _seg0_deg_kernel(h, srcw, dstw):
    def body(h, srcw, dstw, acc_out, deg_out, src_v, dst_v, buf0, buf1,
             acc_sh, sem0, sem1):
        _seg_body(h, srcw, dstw, acc_out, src_v, dst_v, buf0, buf1, acc_sh,
                  sem0, sem1, deg_out=deg_out)

    f = pl.kernel(
        body,
        out_type=[jax.ShapeDtypeStruct((NC, NP, D), _f32),
                  jax.ShapeDtypeStruct((NC, NP, D), _f32)],
        mesh=_sc_mesh(),
        scratch_types=[
            pltpu.VMEM((HCHUNK, CH), jnp.int32),
            pltpu.VMEM((HCHUNK, CH), jnp.int32),
            pltpu.VMEM((CH, D), _f32),
            pltpu.VMEM((CH, D), _f32),
            pltpu.VMEM_SHARED((NP, D), _f32),
            pltpu.SemaphoreType.DMA,
            pltpu.SemaphoreType.DMA,
        ],
    )
    return f(h, srcw, dstw)


R = 2000
GRID = N // R

def _row_spec(width):
    return pl.BlockSpec((R, width), lambda i: (i, 0))


def _full_spec(shape):
    nd = len(shape)
    return pl.BlockSpec(shape, lambda i: (0,) * nd)


def _dot(a, b):
    return jnp.dot(a, b, preferred_element_type=_f32)


def _agg(a0, a1, d0, d1):
    deg = jnp.maximum(d0[:, :1] + d1[:, :1], 1.0)
    return (a0 + a1) / deg


def _conv_out(x, a_r, dg_r, ws, wn, b):
    agg = _agg(a_r[0], a_r[1], dg_r[0], dg_r[1])
    return jnp.maximum(_dot(x, ws) + _dot(agg, wn) + b, 0.0)


def _acc_spec():
    return pl.BlockSpec((NC, R, D), lambda i: (0, i, 0))


def _deg_spec():
    return pl.BlockSpec((NC, R, D), lambda i: (0, i, 0))


def _tc_call(body, n_out, widths_in, widths_out):
    return pl.pallas_call(
        body,
        grid=(GRID,),
        in_specs=[w if isinstance(w, pl.BlockSpec)
                  else (_row_spec(w) if isinstance(w, int) else _full_spec(w))
                  for w in widths_in],
        out_specs=[_row_spec(w) for w in widths_out] if n_out > 1
        else _row_spec(widths_out[0]),
        out_shape=[jax.ShapeDtypeStruct((N, w), _f32) for w in widths_out]
        if n_out > 1 else jax.ShapeDtypeStruct((N, widths_out[0]), _f32),
    )


def _k1(x, ip, t00):
    def body(x_r, w1, b1, g, be, w2, b2, w0, b0, o_r):
        h = jnp.maximum(_dot(x_r[...], w1[...]) + b1[...], 0.0)
        mu = jnp.mean(h, axis=-1, keepdims=True)
        var = jnp.mean(jnp.square(h - mu), axis=-1, keepdims=True)
        hn = (h - mu) * lax.rsqrt(var + 1e-5) * g[...] + be[...]
        t0 = _dot(hn, w2[...]) + b2[...]
        o_r[...] = jnp.maximum(_dot(t0, w0[...]) + b0[...], 0.0)

    mid = ip["l1"]["W"].shape[1]
    f = _tc_call(body, 1,
                 [D, (D, mid), (1, mid), (1, mid), (1, mid), (mid, D), (1, D),
                  (D, D), (1, D)], [D])
    return f(x, ip["l1"]["W"], ip["l1"]["b"].reshape(1, -1),
             ip["g"].reshape(1, -1), ip["be"].reshape(1, -1),
             ip["l2"]["W"], ip["l2"]["b"].reshape(1, -1),
             t00["W"], t00["b"].reshape(1, -1))


def _k2(x, acc, deg, cv):
    def body(x_r, a_r, dg_r, ws, wn, b, o_r):
        o_r[...] = _conv_out(x_r[...], a_r, dg_r, ws[...], wn[...], b[...])

    f = _tc_call(body, 1, [D, _acc_spec(), _deg_spec(), (D, D), (D, D),
                           (1, D)], [D])
    return f(x, acc, deg, cv["Ws"], cv["Wn"], cv["b"].reshape(1, -1))


def _k3(bot, acc, deg, cv, br0, t01):
    def body(bot_r, a_r, dg_r, ws, wn, b, br_r, wt, bt, o_r):
        cb = _conv_out(bot_r[...], a_r, dg_r, ws[...], wn[...], b[...])
        tb = jnp.maximum(_dot(br_r[...], wt[...]) + bt[...], 0.0)
        o_r[...] = (cb + tb) * 0.5

    f = _tc_call(body, 1, [D, _acc_spec(), _deg_spec(), (D, D), (D, D),
                           (1, D), D, (D, D), (1, D)], [D])
    return f(bot, acc, deg, cv["Ws"], cv["Wn"], cv["b"].reshape(1, -1),
             br0, t01["W"], t01["b"].reshape(1, -1))


def _k4(fin, acc, deg, cv, t02, t12):
    def body(f_r, a_r, dg_r, ws, wn, b, w0, b0, w1, b1,
             bot_r, o0_r, o1_r):
        fv = f_r[...]
        bot_r[...] = _conv_out(fv, a_r, dg_r, ws[...], wn[...], b[...])
        o0_r[...] = jnp.maximum(_dot(fv, w0[...]) + b0[...], 0.0)
        o1_r[...] = jnp.maximum(_dot(fv, w1[...]) + b1[...], 0.0)

    f = _tc_call(body, 3, [D, _acc_spec(), _deg_spec(), (D, D), (D, D),
                           (1, D), (D, D), (1, D), (D, D), (1, D)], [D, D, D])
    return f(fin, acc, deg, cv["Ws"], cv["Wn"], cv["b"].reshape(1, -1),
             t02["W"], t02["b"].reshape(1, -1),
             t12["W"], t12["b"].reshape(1, -1))


def _k5(bot, br0, br1, acc, deg, cv, t03, t13):
    def body(bot_r, br0_r, br1_r, a_r, dg_r, ws, wn, b,
             w0, b0, w1, b1, o_r):
        cb = _conv_out(bot_r[...], a_r, dg_r, ws[...], wn[...], b[...])
        tb0 = jnp.maximum(_dot(br0_r[...], w0[...]) + b0[...], 0.0)
        tb1 = jnp.maximum(_dot(br1_r[...], w1[...]) + b1[...], 0.0)
        o_r[...] = (cb + tb0 + tb1) * (1.0 / 3.0)

    f = _tc_call(body, 1, [D, D, D, _acc_spec(), _deg_spec(), (D, D),
                           (D, D), (1, D), (D, D), (1, D), (D, D), (1, D)],
                 [D])
    return f(bot, br0, br1, acc, deg,
             cv["Ws"], cv["Wn"], cv["b"].reshape(1, -1),
             t03["W"], t03["b"].reshape(1, -1),
             t13["W"], t13["b"].reshape(1, -1))


def _k6(f2, acc, deg, cv, t04, t14, t24, dp):
    def body(f_r, a_r, dg_r, ws, wn, b, w0, b0, w1, b1,
             w2, b2, wd, bd, o_r):
        fv = f_r[...]
        cb = _conv_out(fv, a_r, dg_r, ws[...], wn[...], b[...])
        tb0 = jnp.maximum(_dot(fv, w0[...]) + b0[...], 0.0)
        tb1 = jnp.maximum(_dot(fv, w1[...]) + b1[...], 0.0)
        tb2 = jnp.maximum(_dot(fv, w2[...]) + b2[...], 0.0)
        merged = (cb + tb0 + tb1 + tb2) * 0.25
        o_r[...] = _dot(merged, wd[...]) + bd[...]

    f = _tc_call(body, 1, [D, _acc_spec(), _deg_spec(), (D, D), (D, D),
                           (1, D), (D, D), (1, D), (D, D), (1, D),
                           (D, D), (1, D), (D, OUT), (1, OUT)], [OUT])
    return f(f2, acc, deg, cv["Ws"], cv["Wn"], cv["b"].reshape(1, -1),
             t04["W"], t04["b"].reshape(1, -1),
             t14["W"], t14["b"].reshape(1, -1),
             t24["W"], t24["b"].reshape(1, -1),
             dp["W"], dp["b"].reshape(1, -1))


def _prep_edges(edge_index):
    src = edge_index[0]
    dst = edge_index[1]
    pad = E_PAD - E
    ar = jnp.arange(pad, dtype=jnp.int32)
    srcp = jnp.concatenate([src, ar % N])
    dstp = jnp.concatenate([dst, PAD_DST + ar % (NP - N)])
    return (srcp.reshape(NW, NCHUNK, CH), dstp.reshape(NW, NCHUNK, CH))


def kernel(x, edge_index, params):
    conv = params["conv"]
    t = params["t"]
    srcw, dstw = _prep_edges(edge_index)

    br0 = _k1(x, params["ip"], t["0_0"])
    acc, deg = _seg0_deg_kernel(x, srcw, dstw)
    bot = _k2(x, acc, deg, conv[0])

    acc = _seg_kernel(bot, srcw, dstw)
    f = _k3(bot, acc, deg, conv[1], br0, t["0_1"])

    acc = _seg_kernel(f, srcw, dstw)
    bot, br0, br1 = _k4(f, acc, deg, conv[2], t["0_2"], t["1_2"])

    acc = _seg_kernel(bot, srcw, dstw)
    f2 = _k5(bot, br0, br1, acc, deg, conv[3], t["0_3"], t["1_3"])

    acc = _seg_kernel(f2, srcw, dstw)
    return _k6(f2, acc, deg, conv[4], t["0_4"], t["1_4"], t["2_4"],
               params["dp"])
